# Initial kernel scaffold; baseline (speedup 1.0000x reference)
import jax
import jax.numpy as jnp
from jax.experimental import pallas as pl


def kernel(z, tag, z_weight, tag_weight, period_weight, group_weight, properties_mapping, period_mapping, group_mapping):
    raise NotImplementedError("write your pallas kernel here")


# speedup vs baseline: 3.4877x; 3.4877x over previous
import functools

import jax
import jax.numpy as jnp
from jax import lax
from jax.experimental import pallas as pl
from jax.experimental.pallas import tpu as pltpu
from jax.experimental.pallas import tpu_sc as plsc

N_ATOMS = 100000
N_ELEMENTS = 85
N_PROPS = 14
EMB = 32
D_OUT = 142

NUM_WORKERS = 32
CHUNK = 160
N_CHUNKS = N_ATOMS // CHUNK
GROUPS = CHUNK // 16
CHUNKS_PER_W = -(-N_CHUNKS // NUM_WORKERS)

_mesh = plsc.VectorSubcoreMesh(core_axis_name="c", subcore_axis_name="s")


@functools.partial(
    pl.kernel,
    out_type=jax.ShapeDtypeStruct((N_ATOMS, D_OUT), jnp.float32),
    mesh=_mesh,
    scratch_types=[
        pltpu.VMEM((N_ELEMENTS, EMB), jnp.float32),
        pltpu.VMEM((3, EMB), jnp.float32),
        pltpu.VMEM((8, EMB), jnp.float32),
        pltpu.VMEM((20, EMB), jnp.float32),
        pltpu.VMEM((N_ELEMENTS + 1, N_PROPS), jnp.float32),
        pltpu.VMEM((N_ELEMENTS + 1,), jnp.int32),
        pltpu.VMEM((N_ELEMENTS + 1,), jnp.int32),
        pltpu.VMEM((256, D_OUT), jnp.float32),
        pltpu.VMEM((CHUNK,), jnp.int32),
        pltpu.VMEM((CHUNK,), jnp.int32),
        pltpu.VMEM((CHUNK, D_OUT), jnp.float32),
    ],
    compiler_params=pltpu.CompilerParams(needs_layout_passes=False, use_tc_tiling_on_sc=False),
)
def _phys_embedding_sc(z_hbm, tag_hbm, zw_hbm, tw_hbm, pw_hbm, gw_hbm,
                       props_hbm, pmap_hbm, gmap_hbm, out_hbm,
                       zw, tw, pw, gw, props, pmap, gmap, fused,
                       zbuf, tbuf, obuf):
    wid = lax.axis_index("s") * 2 + lax.axis_index("c")

    pltpu.sync_copy(zw_hbm, zw)
    pltpu.sync_copy(tw_hbm, tw)
    pltpu.sync_copy(pw_hbm, pw)
    pltpu.sync_copy(gw_hbm, gw)
    pltpu.sync_copy(props_hbm, props)
    pltpu.sync_copy(pmap_hbm, pmap)
    pltpu.sync_copy(gmap_hbm, gmap)

    lanes = lax.iota(jnp.int32, 16)

    def build_body(i, carry):
        rvec = i * 16 + lanes
        zvec = jnp.minimum(rvec // 3, N_ELEMENTS - 1)
        tvec = rvec - (rvec // 3) * 3
        pvec = plsc.load_gather(pmap, [zvec])
        gvec = plsc.load_gather(gmap, [zvec])
        for c in range(D_OUT):
            cb = jnp.full((16,), c, jnp.int32)
            if c < 32:
                val = plsc.load_gather(zw, [zvec, cb])
            elif c < 46:
                val = plsc.load_gather(props, [zvec, jnp.full((16,), c - 32, jnp.int32)])
            elif c < 78:
                val = plsc.load_gather(pw, [pvec, jnp.full((16,), c - 46, jnp.int32)])
            elif c < 110:
                val = plsc.load_gather(gw, [gvec, jnp.full((16,), c - 78, jnp.int32)])
            else:
                val = plsc.load_gather(tw, [tvec, jnp.full((16,), c - 110, jnp.int32)])
            plsc.store_scatter(fused, [rvec, cb], val)
        return carry

    lax.fori_loop(0, 256 // 16, build_body, 0)

    def chunk_body(ci, carry):
        chunk_id = ci * NUM_WORKERS + wid

        @pl.when(chunk_id < N_CHUNKS)
        def _():
            base = chunk_id * CHUNK
            pltpu.sync_copy(z_hbm.at[pl.ds(base, CHUNK)], zbuf)
            pltpu.sync_copy(tag_hbm.at[pl.ds(base, CHUNK)], tbuf)

            cidx = []
            for g in range(GROUPS):
                zv = zbuf[pl.ds(g * 16, 16)]
                tv = tbuf[pl.ds(g * 16, 16)]
                cidx.append(zv * 3 + tv)

            for c in range(D_OUT):
                cb = jnp.full((16,), c, jnp.int32)
                for g in range(GROUPS):
                    val = plsc.load_gather(fused, [cidx[g], cb])
                    plsc.store_scatter(obuf, [g * 16 + lanes, cb], val)
            pltpu.sync_copy(obuf, out_hbm.at[pl.ds(base, CHUNK)])

        return carry

    lax.fori_loop(0, CHUNKS_PER_W, chunk_body, 0)


def kernel(z, tag, z_weight, tag_weight, period_weight, group_weight,
           properties_mapping, period_mapping, group_mapping):
    return _phys_embedding_sc(z, tag, z_weight, tag_weight, period_weight,
                              group_weight, properties_mapping,
                              period_mapping, group_mapping)
